# baseline (speedup 1.0000x reference)
import jax
import jax.numpy as jnp
from jax.experimental import pallas as pl
from jax.experimental.pallas import tpu as pltpu

B, N, DX, DT, DOUT = 8, 2048, 4, 128, 256
H = N // 2
BLK = 128
STEPS = 16


def _body(t_ref, mask_ref, w1_ref, w2_ref, out_ref, tv_ref, id_ref):
    i = pl.program_id(0)

    @pl.when(i == 0)
    def _init():
        tv_ref[...] = jnp.zeros_like(tv_ref)
        id_ref[...] = jnp.zeros_like(id_ref)

    b = i // 2
    h = i % 2
    mi = mask_ref[pl.ds(b, 1), pl.ds(h * H, H)]
    msqi = mi * mi
    tv_ref[pl.ds(b, 1), :] += jnp.dot(
        msqi, t_ref[0], preferred_element_type=jnp.float32
    )

    mc = mask_ref[:, pl.ds(i * BLK, BLK)]
    id_ref[...] += jnp.dot(
        mc * mc, w2_ref[...], preferred_element_type=jnp.float32
    )

    @pl.when(i == STEPS - 1)
    def _finish():
        m = mask_ref[...]
        denom = jnp.maximum(jnp.sum(m, axis=1, keepdims=True), 1.0)
        out_ref[...] = (
            jnp.dot(tv_ref[...] / denom, w1_ref[...],
                    preferred_element_type=jnp.float32)
            + id_ref[...] / denom
        )


def kernel(x, t, mask, W):
    del x
    mask2d = jnp.reshape(mask, (B, N))
    return pl.pallas_call(
        _body,
        grid=(STEPS,),
        in_specs=[
            pl.BlockSpec((1, H, DT), lambda i: (i // 2, i % 2, 0)),
            pl.BlockSpec((B, N), lambda i: (0, 0)),
            pl.BlockSpec((BLK, DOUT), lambda i: (0, 0)),
            pl.BlockSpec((BLK, DOUT), lambda i: (i + 1, 0)),
        ],
        out_specs=pl.BlockSpec((B, DOUT), lambda i: (0, 0)),
        out_shape=jax.ShapeDtypeStruct((B, DOUT), jnp.float32),
        scratch_shapes=[
            pltpu.VMEM((B, DT), jnp.float32),
            pltpu.VMEM((B, DOUT), jnp.float32),
        ],
    )(t, mask2d, W, W)


# speedup vs baseline: 1.4928x; 1.4928x over previous
import jax
import jax.numpy as jnp
from jax.experimental import pallas as pl
from jax.experimental.pallas import tpu as pltpu

B, N, DX, DT, DOUT = 8, 2048, 4, 128, 256
H = N // 2
BLK = 128
C = N // B


def _body(ta_ref, tb_ref, mask_ref, w1_ref, w2a_ref, w2b_ref, out_ref,
          tv_ref, id_ref):
    i = pl.program_id(0)

    @pl.when(i == 0)
    def _init():
        id_ref[...] = jnp.zeros_like(id_ref)

    mi = mask_ref[pl.ds(i, 1), :]
    msqi = mi * mi
    tv_ref[pl.ds(i, 1), :] = (
        jnp.dot(msqi[:, :H], ta_ref[0], preferred_element_type=jnp.float32)
        + jnp.dot(msqi[:, H:], tb_ref[0], preferred_element_type=jnp.float32)
    )

    ma = mask_ref[:, pl.ds(i * C, BLK)]
    mb = mask_ref[:, pl.ds(i * C + BLK, BLK)]
    id_ref[...] += (
        jnp.dot(ma * ma, w2a_ref[...], preferred_element_type=jnp.float32)
        + jnp.dot(mb * mb, w2b_ref[...], preferred_element_type=jnp.float32)
    )

    @pl.when(i == B - 1)
    def _finish():
        m = mask_ref[...]
        denom = jnp.maximum(jnp.sum(m, axis=1, keepdims=True), 1.0)
        out_ref[...] = (
            jnp.dot(tv_ref[...] / denom, w1_ref[...],
                    preferred_element_type=jnp.float32)
            + id_ref[...] / denom
        )


def kernel(x, t, mask, W):
    del x
    mask2d = jnp.reshape(mask, (B, N))
    return pl.pallas_call(
        _body,
        grid=(B,),
        in_specs=[
            pl.BlockSpec((1, H, DT), lambda i: (i, 0, 0)),
            pl.BlockSpec((1, H, DT), lambda i: (i, 1, 0)),
            pl.BlockSpec((B, N), lambda i: (0, 0)),
            pl.BlockSpec((BLK, DOUT), lambda i: (0, 0)),
            pl.BlockSpec((BLK, DOUT), lambda i: (2 * i + 1, 0)),
            pl.BlockSpec((BLK, DOUT), lambda i: (2 * i + 2, 0)),
        ],
        out_specs=pl.BlockSpec((B, DOUT), lambda i: (0, 0)),
        out_shape=jax.ShapeDtypeStruct((B, DOUT), jnp.float32),
        scratch_shapes=[
            pltpu.VMEM((B, DT), jnp.float32),
            pltpu.VMEM((B, DOUT), jnp.float32),
        ],
    )(t, t, mask2d, W, W, W)
